# Initial kernel scaffold; baseline (speedup 1.0000x reference)
import jax
import jax.numpy as jnp
from jax.experimental import pallas as pl


def kernel(xyz, Z, body23, emb, R_W0, R_b0, R_Wh, R_bh, R_Wf, R_bf, res_W, res_b, W1, b1, bn_g, bn_b, W_out, b_out):
    raise NotImplementedError("write your pallas kernel here")


# speedup vs baseline: 7.0089x; 7.0089x over previous
import math

import jax
import jax.numpy as jnp
import numpy as np
from jax.experimental import pallas as pl
from jax.experimental.pallas import tpu as pltpu

B = 32
N = 30
EMB = 32
CD = 24
CORD = 3
RAD = 2.0
NB = 3
H = 100
LRAD = 5
CLOUD_OUT = CD * CORD ** 2 + 3
RES_OUT = 2 * CLOUD_OUT
FF1 = 128

NBATCH = 4
GRID = B // NBATCH
ROWS = NBATCH * N
E = ROWS * N

_S3 = math.sqrt(3.0)


def _sp5(x):
    return jax.nn.softplus(5.0 * x) * 0.2


def _perm219():
    perm = [0, 1, 2]
    base = 3
    perm += [base + o for o in range(CD)]
    for m in range(3):
        for o in range(CD):
            perm.append(base + CD + o * 3 + m)
    for m in range(5):
        for o in range(CD):
            perm.append(base + CD * 4 + o * 5 + m)
    return np.asarray(perm, dtype=np.int32)


def _se3_kernel(xyz_ref, zoh_ref, body_ref,
                w0_ref, b0_ref, wh_ref, bh_ref,
                a2_ref, bemb_ref,
                resw_ref, resb_ref, w1_ref, b1_ref,
                bng_ref, bnb_ref, wout_ref, bout_ref,
                out_ref, pooled_ref):
    pid = pl.program_id(0)

    px = xyz_ref[:, 0, :]
    py = xyz_ref[:, 1, :]
    pz = xyz_ref[:, 2, :]
    dx = px[:, None, :] - px[:, :, None]
    dy = py[:, None, :] - py[:, :, None]
    dz = pz[:, None, :] - pz[:, :, None]
    d2 = dx * dx + dy * dy + dz * dz + 1e-12
    dist = jnp.sqrt(d2)
    mask = (dist < RAD).astype(jnp.float32)
    valid = (dist > 1e-4).astype(jnp.float32)
    inv = valid / dist
    ux = dx * inv
    uy = dy * inv
    uz = dz * inv
    y2a = _S3 * ux * uy
    y2b = _S3 * uy * uz
    y2c = (0.5 * (3.0 * uz * uz - 1.0)) * valid
    y2d = _S3 * ux * uz
    y2e = (0.5 * _S3) * (ux * ux - uy * uy)

    def bump(c):
        df = dist - c
        return jnp.where(jnp.abs(df) < 1.0, jnp.cos((0.5 * math.pi) * df), 0.0)

    bas = jnp.stack([bump(0.0), bump(1.0), bump(2.0)], axis=-1)
    x_in = bas.reshape(E, NB)

    h = _sp5(jnp.dot(x_in, w0_ref[...],
                     preferred_element_type=jnp.float32) + b0_ref[...])
    for l in range(LRAD - 1):
        h = _sp5(jnp.dot(h, wh_ref[l],
                         preferred_element_type=jnp.float32) + bh_ref[l])
    e3 = jnp.zeros((ROWS, N, CORD * CD), dtype=jnp.float32)
    for z in range(6):
        zrow = zoh_ref[:, z, :]
        zmat = jnp.broadcast_to(zrow[:, None, :], (NBATCH, N, N))
        zflat = zmat.reshape(ROWS, N)[:, :, None]
        ez = jnp.dot(h, a2_ref[:, z * 72:(z + 1) * 72],
                     preferred_element_type=jnp.float32).reshape(ROWS, N, 72)
        e3 = e3 + zflat * (ez + bemb_ref[z][None, None, :])

    def wb(w):
        return jnp.broadcast_to(w.reshape(ROWS, N)[:, :, None], (ROWS, N, CD))

    wcat = jnp.concatenate(
        [wb(mask), wb(ux * mask), wb(uy * mask), wb(uz * mask),
         wb(y2a * mask), wb(y2b * mask), wb(y2c * mask), wb(y2d * mask),
         wb(y2e * mask)], axis=-1)
    e0 = e3[:, :, 0 * CD:1 * CD]
    e1 = e3[:, :, 1 * CD:2 * CD]
    e2 = e3[:, :, 2 * CD:3 * CD]
    ecat = jnp.concatenate([e0, e1, e1, e1, e2, e2, e2, e2, e2], axis=-1)
    feats_m = jnp.sum(wcat * ecat, axis=1)

    body = body_ref[...].reshape(ROWS, 3)
    feats = jnp.concatenate([body, feats_m], axis=-1)
    hres = feats + jax.nn.relu(
        jnp.dot(feats, resw_ref[...],
                preferred_element_type=jnp.float32) + resb_ref[...])
    feats2 = jnp.concatenate([feats, hres], axis=-1)

    pooled = jnp.sum(feats2.reshape(NBATCH, N, RES_OUT), axis=1)
    pooled_ref[pid] = pooled

    @pl.when(pid == GRID - 1)
    def _():
        pall = pooled_ref[...].reshape(B, RES_OUT)
        x = jax.nn.softplus(
            jnp.dot(pall, w1_ref[...],
                    preferred_element_type=jnp.float32) + b1_ref[...])
        mean = jnp.mean(x, axis=0, keepdims=True)
        var = jnp.mean((x - mean) * (x - mean), axis=0, keepdims=True)
        xn = jax.nn.softplus(
            bng_ref[...] * (x - mean) * jax.lax.rsqrt(var + 1e-5)
            + bnb_ref[...])
        out_ref[...] = jax.nn.sigmoid(
            jnp.dot(xn, wout_ref[...],
                    preferred_element_type=jnp.float32) + bout_ref[...])


def kernel(xyz, Z, body23, emb, R_W0, R_b0, R_Wh, R_bh, R_Wf, R_bf,
           res_W, res_b, W1, b1, bn_g, bn_b, W_out, b_out):
    f32 = jnp.float32
    wf4 = R_Wf.reshape(H, CORD, CD, EMB)
    a2 = jnp.einsum('klmc,zc->kzlm', wf4, emb).reshape(H, 6 * CORD * CD)
    bemb = jnp.einsum('lmc,zc->zlm', R_bf.reshape(CORD, CD, EMB),
                      emb).reshape(6, CORD * CD)
    zoh = jax.nn.one_hot(Z, 6, dtype=f32).transpose(0, 2, 1)
    xyz_t = xyz.transpose(0, 2, 1)

    p219 = _perm219()
    p438 = np.concatenate([p219, p219 + CLOUD_OUT])
    res_w_p = res_W[p219][:, p219]
    res_b_p = res_b[p219].reshape(1, CLOUD_OUT)
    w1_p = W1[p438, :]

    full = lambda a: pl.BlockSpec(a.shape, lambda ib: (0,) * a.ndim)
    out = pl.pallas_call(
        _se3_kernel,
        grid=(GRID,),
        in_specs=[
            pl.BlockSpec((NBATCH, 3, N), lambda ib: (ib, 0, 0)),
            pl.BlockSpec((NBATCH, 6, N), lambda ib: (ib, 0, 0)),
            pl.BlockSpec((NBATCH, N, 3), lambda ib: (ib, 0, 0)),
            full(R_W0),
            pl.BlockSpec((1, H), lambda ib: (0, 0)),
            full(R_Wh),
            pl.BlockSpec((LRAD - 1, 1, H), lambda ib: (0, 0, 0)),
            full(a2),
            full(bemb),
            full(res_w_p),
            full(res_b_p),
            full(w1_p),
            pl.BlockSpec((1, FF1), lambda ib: (0, 0)),
            pl.BlockSpec((1, FF1), lambda ib: (0, 0)),
            pl.BlockSpec((1, FF1), lambda ib: (0, 0)),
            full(W_out),
            pl.BlockSpec((1, 1), lambda ib: (0, 0)),
        ],
        out_specs=pl.BlockSpec((B, 1), lambda ib: (0, 0)),
        out_shape=jax.ShapeDtypeStruct((B, 1), f32),
        scratch_shapes=[pltpu.VMEM((GRID, NBATCH, RES_OUT), f32)],
    )(xyz_t, zoh, body23,
      R_W0, R_b0.reshape(1, H), R_Wh, R_bh.reshape(LRAD - 1, 1, H),
      a2, bemb, res_w_p, res_b_p, w1_p, b1.reshape(1, FF1),
      bn_g.reshape(1, FF1), bn_b.reshape(1, FF1), W_out,
      b_out.reshape(1, 1))
    return out


# speedup vs baseline: 8.6648x; 1.2363x over previous
import math

import jax
import jax.numpy as jnp
import numpy as np
from jax.experimental import pallas as pl
from jax.experimental.pallas import tpu as pltpu

B = 32
N = 30
EMB = 32
CD = 24
CORD = 3
RAD = 2.0
NB = 3
H = 100
LRAD = 5
CLOUD_OUT = CD * CORD ** 2 + 3
RES_OUT = 2 * CLOUD_OUT
FF1 = 128

NBATCH = 4
GRID = B // NBATCH
NP = 32
ROWS = NBATCH * N
E = ROWS * NP

_S3 = math.sqrt(3.0)


def _sp5(x):
    return jax.nn.softplus(5.0 * x) * 0.2


def _perm219():
    perm = [0, 1, 2]
    base = 3
    perm += [base + o for o in range(CD)]
    for m in range(3):
        for o in range(CD):
            perm.append(base + CD + o * 3 + m)
    for m in range(5):
        for o in range(CD):
            perm.append(base + CD * 4 + o * 5 + m)
    return np.asarray(perm, dtype=np.int32)


def _se3_kernel(xyz_ref, zoh_ref, body_ref,
                w0_ref, b0_ref, wh_ref, bh_ref,
                a2_ref, bemb_ref,
                resw_ref, resb_ref, w1_ref, b1_ref,
                bng_ref, bnb_ref, wout_ref, bout_ref,
                out_ref, pooled_ref):
    pid = pl.program_id(0)

    px = xyz_ref[:, 0, :]
    py = xyz_ref[:, 1, :]
    pz = xyz_ref[:, 2, :]
    dx = px[:, None, :] - px[:, :N, None]
    dy = py[:, None, :] - py[:, :N, None]
    dz = pz[:, None, :] - pz[:, :N, None]
    d2 = dx * dx + dy * dy + dz * dz + 1e-12
    dist = jnp.sqrt(d2)
    jreal = jax.lax.broadcasted_iota(jnp.int32, (NBATCH, N, NP), 2) < N
    mask = jnp.where(jreal, (dist < RAD).astype(jnp.float32), 0.0)
    valid = (dist > 1e-4).astype(jnp.float32)
    inv = valid / dist
    ux = dx * inv
    uy = dy * inv
    uz = dz * inv
    y2a = _S3 * ux * uy
    y2b = _S3 * uy * uz
    y2c = (0.5 * (3.0 * uz * uz - 1.0)) * valid
    y2d = _S3 * ux * uz
    y2e = (0.5 * _S3) * (ux * ux - uy * uy)

    def bump(c):
        df = dist - c
        return jnp.where(jnp.abs(df) < 1.0, jnp.cos((0.5 * math.pi) * df), 0.0)

    bas = jnp.stack([bump(0.0), bump(1.0), bump(2.0)], axis=-1)
    x_in = bas.reshape(E, NB)

    h = _sp5(jnp.dot(x_in, w0_ref[...],
                     preferred_element_type=jnp.float32) + b0_ref[...])
    for l in range(LRAD - 1):
        h = _sp5(jnp.dot(h, wh_ref[l],
                         preferred_element_type=jnp.float32) + bh_ref[l])
    e3 = jnp.zeros((ROWS, NP, CORD * CD), dtype=jnp.float32)
    for z in range(6):
        zrow = zoh_ref[:, z, :]
        zmat = jnp.broadcast_to(zrow[:, None, :], (NBATCH, N, NP))
        zflat = zmat.reshape(ROWS, NP)[:, :, None]
        ez = jnp.dot(h, a2_ref[:, z * 72:(z + 1) * 72],
                     preferred_element_type=jnp.float32).reshape(ROWS, NP, 72)
        e3 = e3 + zflat * (ez + bemb_ref[z][None, None, :])

    def wb(w):
        return jnp.broadcast_to(w.reshape(ROWS, NP)[:, :, None], (ROWS, NP, CD))

    wcat = jnp.concatenate(
        [wb(mask), wb(ux * mask), wb(uy * mask), wb(uz * mask),
         wb(y2a * mask), wb(y2b * mask), wb(y2c * mask), wb(y2d * mask),
         wb(y2e * mask)], axis=-1)
    e0 = e3[:, :, 0 * CD:1 * CD]
    e1 = e3[:, :, 1 * CD:2 * CD]
    e2 = e3[:, :, 2 * CD:3 * CD]
    ecat = jnp.concatenate([e0, e1, e1, e1, e2, e2, e2, e2, e2], axis=-1)
    feats_m = jnp.sum(wcat * ecat, axis=1)

    body = body_ref[...].reshape(ROWS, 3)
    feats = jnp.concatenate([body, feats_m], axis=-1)
    hres = feats + jax.nn.relu(
        jnp.dot(feats, resw_ref[...],
                preferred_element_type=jnp.float32) + resb_ref[...])
    feats2 = jnp.concatenate([feats, hres], axis=-1)

    pooled = jnp.sum(feats2.reshape(NBATCH, N, RES_OUT), axis=1)
    pooled_ref[pid] = pooled

    @pl.when(pid == GRID - 1)
    def _():
        pall = pooled_ref[...].reshape(B, RES_OUT)
        x = jax.nn.softplus(
            jnp.dot(pall, w1_ref[...],
                    preferred_element_type=jnp.float32) + b1_ref[...])
        mean = jnp.mean(x, axis=0, keepdims=True)
        var = jnp.mean((x - mean) * (x - mean), axis=0, keepdims=True)
        xn = jax.nn.softplus(
            bng_ref[...] * (x - mean) * jax.lax.rsqrt(var + 1e-5)
            + bnb_ref[...])
        out_ref[...] = jax.nn.sigmoid(
            jnp.dot(xn, wout_ref[...],
                    preferred_element_type=jnp.float32) + bout_ref[...])


def kernel(xyz, Z, body23, emb, R_W0, R_b0, R_Wh, R_bh, R_Wf, R_bf,
           res_W, res_b, W1, b1, bn_g, bn_b, W_out, b_out):
    f32 = jnp.float32
    wf4 = R_Wf.reshape(H, CORD, CD, EMB)
    a2 = jnp.einsum('klmc,zc->kzlm', wf4, emb).reshape(H, 6 * CORD * CD)
    bemb = jnp.einsum('lmc,zc->zlm', R_bf.reshape(CORD, CD, EMB),
                      emb).reshape(6, CORD * CD)
    pad_j = [(0, 0), (0, 0), (0, NP - N)]
    zoh = jnp.pad(jax.nn.one_hot(Z, 6, dtype=f32).transpose(0, 2, 1), pad_j)
    xyz_t = jnp.pad(xyz.transpose(0, 2, 1), pad_j)

    p219 = _perm219()
    p438 = np.concatenate([p219, p219 + CLOUD_OUT])
    res_w_p = res_W[p219][:, p219]
    res_b_p = res_b[p219].reshape(1, CLOUD_OUT)
    w1_p = W1[p438, :]

    full = lambda a: pl.BlockSpec(a.shape, lambda ib: (0,) * a.ndim)
    out = pl.pallas_call(
        _se3_kernel,
        grid=(GRID,),
        in_specs=[
            pl.BlockSpec((NBATCH, 3, NP), lambda ib: (ib, 0, 0)),
            pl.BlockSpec((NBATCH, 6, NP), lambda ib: (ib, 0, 0)),
            pl.BlockSpec((NBATCH, N, 3), lambda ib: (ib, 0, 0)),
            full(R_W0),
            pl.BlockSpec((1, H), lambda ib: (0, 0)),
            full(R_Wh),
            pl.BlockSpec((LRAD - 1, 1, H), lambda ib: (0, 0, 0)),
            full(a2),
            full(bemb),
            full(res_w_p),
            full(res_b_p),
            full(w1_p),
            pl.BlockSpec((1, FF1), lambda ib: (0, 0)),
            pl.BlockSpec((1, FF1), lambda ib: (0, 0)),
            pl.BlockSpec((1, FF1), lambda ib: (0, 0)),
            full(W_out),
            pl.BlockSpec((1, 1), lambda ib: (0, 0)),
        ],
        out_specs=pl.BlockSpec((B, 1), lambda ib: (0, 0)),
        out_shape=jax.ShapeDtypeStruct((B, 1), f32),
        scratch_shapes=[pltpu.VMEM((GRID, NBATCH, RES_OUT), f32)],
    )(xyz_t, zoh, body23,
      R_W0, R_b0.reshape(1, H), R_Wh, R_bh.reshape(LRAD - 1, 1, H),
      a2, bemb, res_w_p, res_b_p, w1_p, b1.reshape(1, FF1),
      bn_g.reshape(1, FF1), bn_b.reshape(1, FF1), W_out,
      b_out.reshape(1, 1))
    return out


# speedup vs baseline: 10.5079x; 1.2127x over previous
import math

import jax
import jax.numpy as jnp
import numpy as np
from jax.experimental import pallas as pl
from jax.experimental.pallas import tpu as pltpu

B = 32
N = 30
EMB = 32
CD = 24
CORD = 3
RAD = 2.0
NB = 3
H = 100
LRAD = 5
CLOUD_OUT = CD * CORD ** 2 + 3
RES_OUT = 2 * CLOUD_OUT
FF1 = 128

NBATCH = 4
GRID = B // NBATCH
NP = 32
ROWS = NBATCH * N
E = ROWS * NP

_S3 = math.sqrt(3.0)


def _sp5(x):
    return jax.nn.softplus(5.0 * x) * 0.2


def _perm219():
    perm = [0, 1, 2]
    base = 3
    perm += [base + o for o in range(CD)]
    for m in range(3):
        for o in range(CD):
            perm.append(base + CD + o * 3 + m)
    for m in range(5):
        for o in range(CD):
            perm.append(base + CD * 4 + o * 5 + m)
    return np.asarray(perm, dtype=np.int32)


def _se3_kernel(xyz_ref, zoh_ref, body_ref,
                w0_ref, b0_ref, wh_ref, bh_ref,
                a2_ref,
                resw_ref, resb_ref, w1_ref, b1_ref,
                bng_ref, bnb_ref, wout_ref, bout_ref,
                out_ref, pooled_ref):
    pid = pl.program_id(0)

    px = xyz_ref[:, 0, :]
    py = xyz_ref[:, 1, :]
    pz = xyz_ref[:, 2, :]
    dx = px[:, None, :] - px[:, :N, None]
    dy = py[:, None, :] - py[:, :N, None]
    dz = pz[:, None, :] - pz[:, :N, None]
    d2 = dx * dx + dy * dy + dz * dz + 1e-12
    dist = jnp.sqrt(d2)
    jreal = jax.lax.broadcasted_iota(jnp.int32, (NBATCH, N, NP), 2) < N
    mask = jnp.where(jreal, (dist < RAD).astype(jnp.float32), 0.0)
    valid = (dist > 1e-4).astype(jnp.float32)
    inv = valid / dist
    ux = dx * inv
    uy = dy * inv
    uz = dz * inv
    y2a = _S3 * ux * uy
    y2b = _S3 * uy * uz
    y2c = (0.5 * (3.0 * uz * uz - 1.0)) * valid
    y2d = _S3 * ux * uz
    y2e = (0.5 * _S3) * (ux * ux - uy * uy)

    def bump(c):
        df = dist - c
        return jnp.where(jnp.abs(df) < 1.0, jnp.cos((0.5 * math.pi) * df), 0.0)

    bas = jnp.stack([bump(0.0), bump(1.0), bump(2.0)], axis=-1)
    x_in = bas.reshape(E, NB)

    h = _sp5(jnp.dot(x_in, w0_ref[...],
                     preferred_element_type=jnp.float32) + b0_ref[...])
    for l in range(LRAD - 1):
        h = _sp5(jnp.dot(h, wh_ref[l],
                         preferred_element_type=jnp.float32) + bh_ref[l])
    lane = jax.lax.broadcasted_iota(jnp.int32, (E, 128), 1)
    hpad = jnp.concatenate([h, jnp.zeros((E, 128 - H), jnp.float32)], axis=1)
    h1p = jnp.where(lane == H, 1.0, hpad)
    h3 = h1p.reshape(ROWS, NP, 128)
    blocks = []
    for z in range(6):
        zrow = zoh_ref[:, z, :]
        zmat = jnp.broadcast_to(zrow[:, None, :], (NBATCH, N, NP))
        blocks.append(h3 * zmat.reshape(ROWS, NP)[:, :, None])
    hz = jnp.concatenate(blocks, axis=-1).reshape(E, 768)
    e3 = jnp.dot(hz, a2_ref[...],
                 preferred_element_type=jnp.float32).reshape(ROWS, NP, 72)

    e0 = e3[:, :, 0 * CD:1 * CD]
    e1 = e3[:, :, 1 * CD:2 * CD]
    e2 = e3[:, :, 2 * CD:3 * CD]
    pairs = ((mask, e0), (ux * mask, e1), (uy * mask, e1), (uz * mask, e1),
             (y2a * mask, e2), (y2b * mask, e2), (y2c * mask, e2),
             (y2d * mask, e2), (y2e * mask, e2))
    feats_m = jnp.concatenate(
        [jnp.sum(w.reshape(ROWS, NP)[:, :, None] * ev, axis=1)
         for w, ev in pairs], axis=-1)

    body = body_ref[...].reshape(ROWS, 3)
    feats = jnp.concatenate([body, feats_m], axis=-1)
    hres = feats + jax.nn.relu(
        jnp.dot(feats, resw_ref[...],
                preferred_element_type=jnp.float32) + resb_ref[...])
    feats2 = jnp.concatenate([feats, hres], axis=-1)

    pooled = jnp.sum(feats2.reshape(NBATCH, N, RES_OUT), axis=1)
    pooled_ref[pid] = pooled

    @pl.when(pid == GRID - 1)
    def _():
        pall = pooled_ref[...].reshape(B, RES_OUT)
        x = jax.nn.softplus(
            jnp.dot(pall, w1_ref[...],
                    preferred_element_type=jnp.float32) + b1_ref[...])
        mean = jnp.mean(x, axis=0, keepdims=True)
        var = jnp.mean((x - mean) * (x - mean), axis=0, keepdims=True)
        xn = jax.nn.softplus(
            bng_ref[...] * (x - mean) * jax.lax.rsqrt(var + 1e-5)
            + bnb_ref[...])
        out_ref[...] = jax.nn.sigmoid(
            jnp.dot(xn, wout_ref[...],
                    preferred_element_type=jnp.float32) + bout_ref[...])


def kernel(xyz, Z, body23, emb, R_W0, R_b0, R_Wh, R_bh, R_Wf, R_bf,
           res_W, res_b, W1, b1, bn_g, bn_b, W_out, b_out):
    f32 = jnp.float32
    wf4 = R_Wf.reshape(H, CORD, CD, EMB)
    a2 = jnp.einsum('klmc,zc->kzlm', wf4, emb).reshape(H, 6 * CORD * CD)
    bemb = jnp.einsum('lmc,zc->zlm', R_bf.reshape(CORD, CD, EMB),
                      emb).reshape(6, CORD * CD)
    parts = []
    for z in range(6):
        blk = jnp.zeros((128, CORD * CD), f32)
        blk = blk.at[:H].set(a2[:, z * 72:(z + 1) * 72])
        blk = blk.at[H].set(bemb[z])
        parts.append(blk)
    a2z = jnp.concatenate(parts, axis=0)
    pad_j = [(0, 0), (0, 0), (0, NP - N)]
    zoh = jnp.pad(jax.nn.one_hot(Z, 6, dtype=f32).transpose(0, 2, 1), pad_j)
    xyz_t = jnp.pad(xyz.transpose(0, 2, 1), pad_j)

    p219 = _perm219()
    p438 = np.concatenate([p219, p219 + CLOUD_OUT])
    res_w_p = res_W[p219][:, p219]
    res_b_p = res_b[p219].reshape(1, CLOUD_OUT)
    w1_p = W1[p438, :]

    full = lambda a: pl.BlockSpec(a.shape, lambda ib: (0,) * a.ndim)
    out = pl.pallas_call(
        _se3_kernel,
        grid=(GRID,),
        in_specs=[
            pl.BlockSpec((NBATCH, 3, NP), lambda ib: (ib, 0, 0)),
            pl.BlockSpec((NBATCH, 6, NP), lambda ib: (ib, 0, 0)),
            pl.BlockSpec((NBATCH, N, 3), lambda ib: (ib, 0, 0)),
            full(R_W0),
            pl.BlockSpec((1, H), lambda ib: (0, 0)),
            full(R_Wh),
            pl.BlockSpec((LRAD - 1, 1, H), lambda ib: (0, 0, 0)),
            full(a2z),
            full(res_w_p),
            full(res_b_p),
            full(w1_p),
            pl.BlockSpec((1, FF1), lambda ib: (0, 0)),
            pl.BlockSpec((1, FF1), lambda ib: (0, 0)),
            pl.BlockSpec((1, FF1), lambda ib: (0, 0)),
            full(W_out),
            pl.BlockSpec((1, 1), lambda ib: (0, 0)),
        ],
        out_specs=pl.BlockSpec((B, 1), lambda ib: (0, 0)),
        out_shape=jax.ShapeDtypeStruct((B, 1), f32),
        scratch_shapes=[pltpu.VMEM((GRID, NBATCH, RES_OUT), f32)],
    )(xyz_t, zoh, body23,
      R_W0, R_b0.reshape(1, H), R_Wh, R_bh.reshape(LRAD - 1, 1, H),
      a2z, res_w_p, res_b_p, w1_p, b1.reshape(1, FF1),
      bn_g.reshape(1, FF1), bn_b.reshape(1, FF1), W_out,
      b_out.reshape(1, 1))
    return out


# speedup vs baseline: 10.6694x; 1.0154x over previous
import math

import jax
import jax.numpy as jnp
import numpy as np
from jax.experimental import pallas as pl
from jax.experimental.pallas import tpu as pltpu

B = 32
N = 30
EMB = 32
CD = 24
CORD = 3
RAD = 2.0
NB = 3
H = 100
LRAD = 5
CLOUD_OUT = CD * CORD ** 2 + 3
RES_OUT = 2 * CLOUD_OUT
FF1 = 128

NBATCH = 8
GRID = B // NBATCH
NP = 32
ROWS = NBATCH * N
E = ROWS * NP

_S3 = math.sqrt(3.0)


def _sp5(x):
    return jax.nn.softplus(5.0 * x) * 0.2


def _perm219():
    perm = [0, 1, 2]
    base = 3
    perm += [base + o for o in range(CD)]
    for m in range(3):
        for o in range(CD):
            perm.append(base + CD + o * 3 + m)
    for m in range(5):
        for o in range(CD):
            perm.append(base + CD * 4 + o * 5 + m)
    return np.asarray(perm, dtype=np.int32)


def _se3_kernel(xyz_ref, zoh_ref, body_ref,
                w0_ref, b0_ref, wh_ref, bh_ref,
                a2_ref,
                resw_ref, resb_ref, w1_ref, b1_ref,
                bng_ref, bnb_ref, wout_ref, bout_ref,
                out_ref, pooled_ref):
    pid = pl.program_id(0)

    px = xyz_ref[:, 0, :]
    py = xyz_ref[:, 1, :]
    pz = xyz_ref[:, 2, :]
    dx = px[:, None, :] - px[:, :N, None]
    dy = py[:, None, :] - py[:, :N, None]
    dz = pz[:, None, :] - pz[:, :N, None]
    d2 = dx * dx + dy * dy + dz * dz + 1e-12
    dist = jnp.sqrt(d2)
    jreal = jax.lax.broadcasted_iota(jnp.int32, (NBATCH, N, NP), 2) < N
    mask = jnp.where(jreal, (dist < RAD).astype(jnp.float32), 0.0)
    valid = (dist > 1e-4).astype(jnp.float32)
    inv = valid / dist
    ux = dx * inv
    uy = dy * inv
    uz = dz * inv
    y2a = _S3 * ux * uy
    y2b = _S3 * uy * uz
    y2c = (0.5 * (3.0 * uz * uz - 1.0)) * valid
    y2d = _S3 * ux * uz
    y2e = (0.5 * _S3) * (ux * ux - uy * uy)

    def bump(c):
        df = dist - c
        return jnp.where(jnp.abs(df) < 1.0, jnp.cos((0.5 * math.pi) * df), 0.0)

    bas = jnp.stack([bump(0.0), bump(1.0), bump(2.0)], axis=-1)
    x_in = bas.reshape(E, NB)

    h = _sp5(jnp.dot(x_in, w0_ref[...],
                     preferred_element_type=jnp.float32) + b0_ref[...])
    for l in range(LRAD - 1):
        h = _sp5(jnp.dot(h, wh_ref[l],
                         preferred_element_type=jnp.float32) + bh_ref[l])
    lane = jax.lax.broadcasted_iota(jnp.int32, (E, 128), 1)
    hpad = jnp.concatenate([h, jnp.zeros((E, 128 - H), jnp.float32)], axis=1)
    h1p = jnp.where(lane == H, 1.0, hpad)
    h3 = h1p.reshape(ROWS, NP, 128)
    blocks = []
    for z in range(6):
        zrow = zoh_ref[:, z, :]
        zmat = jnp.broadcast_to(zrow[:, None, :], (NBATCH, N, NP))
        blocks.append(h3 * zmat.reshape(ROWS, NP)[:, :, None])
    hz = jnp.concatenate(blocks, axis=-1).reshape(E, 768)
    e3 = jnp.dot(hz, a2_ref[...],
                 preferred_element_type=jnp.float32).reshape(ROWS, NP, 72)

    e0 = e3[:, :, 0 * CD:1 * CD]
    e1 = e3[:, :, 1 * CD:2 * CD]
    e2 = e3[:, :, 2 * CD:3 * CD]
    pairs = ((mask, e0), (ux * mask, e1), (uy * mask, e1), (uz * mask, e1),
             (y2a * mask, e2), (y2b * mask, e2), (y2c * mask, e2),
             (y2d * mask, e2), (y2e * mask, e2))
    feats_m = jnp.concatenate(
        [jnp.sum(w.reshape(ROWS, NP)[:, :, None] * ev, axis=1)
         for w, ev in pairs], axis=-1)

    body = body_ref[...].reshape(ROWS, 3)
    feats = jnp.concatenate([body, feats_m], axis=-1)
    hres = feats + jax.nn.relu(
        jnp.dot(feats, resw_ref[...],
                preferred_element_type=jnp.float32) + resb_ref[...])
    feats2 = jnp.concatenate([feats, hres], axis=-1)

    pooled = jnp.sum(feats2.reshape(NBATCH, N, RES_OUT), axis=1)
    pooled_ref[pid] = pooled

    @pl.when(pid == GRID - 1)
    def _():
        pall = pooled_ref[...].reshape(B, RES_OUT)
        x = jax.nn.softplus(
            jnp.dot(pall, w1_ref[...],
                    preferred_element_type=jnp.float32) + b1_ref[...])
        mean = jnp.mean(x, axis=0, keepdims=True)
        var = jnp.mean((x - mean) * (x - mean), axis=0, keepdims=True)
        xn = jax.nn.softplus(
            bng_ref[...] * (x - mean) * jax.lax.rsqrt(var + 1e-5)
            + bnb_ref[...])
        out_ref[...] = jax.nn.sigmoid(
            jnp.dot(xn, wout_ref[...],
                    preferred_element_type=jnp.float32) + bout_ref[...])


def kernel(xyz, Z, body23, emb, R_W0, R_b0, R_Wh, R_bh, R_Wf, R_bf,
           res_W, res_b, W1, b1, bn_g, bn_b, W_out, b_out):
    f32 = jnp.float32
    wf4 = R_Wf.reshape(H, CORD, CD, EMB)
    a2 = jnp.einsum('klmc,zc->kzlm', wf4, emb).reshape(H, 6 * CORD * CD)
    bemb = jnp.einsum('lmc,zc->zlm', R_bf.reshape(CORD, CD, EMB),
                      emb).reshape(6, CORD * CD)
    parts = []
    for z in range(6):
        blk = jnp.zeros((128, CORD * CD), f32)
        blk = blk.at[:H].set(a2[:, z * 72:(z + 1) * 72])
        blk = blk.at[H].set(bemb[z])
        parts.append(blk)
    a2z = jnp.concatenate(parts, axis=0)
    pad_j = [(0, 0), (0, 0), (0, NP - N)]
    zoh = jnp.pad(jax.nn.one_hot(Z, 6, dtype=f32).transpose(0, 2, 1), pad_j)
    xyz_t = jnp.pad(xyz.transpose(0, 2, 1), pad_j)

    p219 = _perm219()
    p438 = np.concatenate([p219, p219 + CLOUD_OUT])
    res_w_p = res_W[p219][:, p219]
    res_b_p = res_b[p219].reshape(1, CLOUD_OUT)
    w1_p = W1[p438, :]

    full = lambda a: pl.BlockSpec(a.shape, lambda ib: (0,) * a.ndim)
    out = pl.pallas_call(
        _se3_kernel,
        grid=(GRID,),
        in_specs=[
            pl.BlockSpec((NBATCH, 3, NP), lambda ib: (ib, 0, 0)),
            pl.BlockSpec((NBATCH, 6, NP), lambda ib: (ib, 0, 0)),
            pl.BlockSpec((NBATCH, N, 3), lambda ib: (ib, 0, 0)),
            full(R_W0),
            pl.BlockSpec((1, H), lambda ib: (0, 0)),
            full(R_Wh),
            pl.BlockSpec((LRAD - 1, 1, H), lambda ib: (0, 0, 0)),
            full(a2z),
            full(res_w_p),
            full(res_b_p),
            full(w1_p),
            pl.BlockSpec((1, FF1), lambda ib: (0, 0)),
            pl.BlockSpec((1, FF1), lambda ib: (0, 0)),
            pl.BlockSpec((1, FF1), lambda ib: (0, 0)),
            full(W_out),
            pl.BlockSpec((1, 1), lambda ib: (0, 0)),
        ],
        out_specs=pl.BlockSpec((B, 1), lambda ib: (0, 0)),
        out_shape=jax.ShapeDtypeStruct((B, 1), f32),
        scratch_shapes=[pltpu.VMEM((GRID, NBATCH, RES_OUT), f32)],
    )(xyz_t, zoh, body23,
      R_W0, R_b0.reshape(1, H), R_Wh, R_bh.reshape(LRAD - 1, 1, H),
      a2z, res_w_p, res_b_p, w1_p, b1.reshape(1, FF1),
      bn_g.reshape(1, FF1), bn_b.reshape(1, FF1), W_out,
      b_out.reshape(1, 1))
    return out


# speedup vs baseline: 10.9499x; 1.0263x over previous
import math

import jax
import jax.numpy as jnp
import numpy as np
from jax.experimental import pallas as pl
from jax.experimental.pallas import tpu as pltpu

B = 32
N = 30
EMB = 32
CD = 24
CORD = 3
RAD = 2.0
NB = 3
H = 100
LRAD = 5
CLOUD_OUT = CD * CORD ** 2 + 3
RES_OUT = 2 * CLOUD_OUT
FF1 = 128

NBATCH = 8
GRID = B // NBATCH
NP = 32
ROWS = NBATCH * N
E = ROWS * NP

_S3 = math.sqrt(3.0)


def _sp5(x):
    return jax.nn.softplus(x)


def _perm219():
    perm = [0, 1, 2]
    base = 3
    perm += [base + o for o in range(CD)]
    for m in range(3):
        for o in range(CD):
            perm.append(base + CD + o * 3 + m)
    for m in range(5):
        for o in range(CD):
            perm.append(base + CD * 4 + o * 5 + m)
    return np.asarray(perm, dtype=np.int32)


def _se3_kernel(xyz_ref, zoh_ref, body_ref,
                w0_ref, b0_ref, wh_ref, bh_ref,
                a2_ref,
                resw_ref, resb_ref, w1_ref, b1_ref,
                bng_ref, bnb_ref, wout_ref, bout_ref,
                out_ref, pooled_ref):
    pid = pl.program_id(0)

    px = xyz_ref[:, 0, :]
    py = xyz_ref[:, 1, :]
    pz = xyz_ref[:, 2, :]
    dx = px[:, None, :] - px[:, :N, None]
    dy = py[:, None, :] - py[:, :N, None]
    dz = pz[:, None, :] - pz[:, :N, None]
    d2 = dx * dx + dy * dy + dz * dz + 1e-12
    dist = jnp.sqrt(d2)
    jreal = jax.lax.broadcasted_iota(jnp.int32, (NBATCH, N, NP), 2) < N
    mask = jnp.where(jreal, (dist < RAD).astype(jnp.float32), 0.0)
    valid = (dist > 1e-4).astype(jnp.float32)
    inv = valid / dist
    ux = dx * inv
    uy = dy * inv
    uz = dz * inv
    y2a = _S3 * ux * uy
    y2b = _S3 * uy * uz
    y2c = (0.5 * (3.0 * uz * uz - 1.0)) * valid
    y2d = _S3 * ux * uz
    y2e = (0.5 * _S3) * (ux * ux - uy * uy)

    def bump(c):
        df = dist - c
        return jnp.where(jnp.abs(df) < 1.0, jnp.cos((0.5 * math.pi) * df), 0.0)

    bas = jnp.stack([bump(0.0), bump(1.0), bump(2.0)], axis=-1)
    x_in = bas.reshape(E, NB)

    h = _sp5(jnp.dot(x_in, w0_ref[...],
                     preferred_element_type=jnp.float32) + b0_ref[...])
    for l in range(LRAD - 1):
        h = _sp5(jnp.dot(h, wh_ref[l],
                         preferred_element_type=jnp.float32) + bh_ref[l])
    lane = jax.lax.broadcasted_iota(jnp.int32, (E, 128), 1)
    hpad = jnp.concatenate([h, jnp.zeros((E, 128 - H), jnp.float32)], axis=1)
    h1p = jnp.where(lane == H, 1.0, hpad)
    h3 = h1p.reshape(ROWS, NP, 128)
    blocks = []
    for z in range(6):
        zrow = zoh_ref[:, z, :]
        zmat = jnp.broadcast_to(zrow[:, None, :], (NBATCH, N, NP))
        blocks.append(h3 * zmat.reshape(ROWS, NP)[:, :, None])
    hz = jnp.concatenate(blocks, axis=-1).reshape(E, 768)
    e3 = jnp.dot(hz, a2_ref[...],
                 preferred_element_type=jnp.float32).reshape(ROWS, NP, 72)

    e0 = e3[:, :, 0 * CD:1 * CD]
    e1 = e3[:, :, 1 * CD:2 * CD]
    e2 = e3[:, :, 2 * CD:3 * CD]
    pairs = ((mask, e0), (ux * mask, e1), (uy * mask, e1), (uz * mask, e1),
             (y2a * mask, e2), (y2b * mask, e2), (y2c * mask, e2),
             (y2d * mask, e2), (y2e * mask, e2))
    feats_m = jnp.concatenate(
        [jnp.sum(w.reshape(ROWS, NP)[:, :, None] * ev, axis=1)
         for w, ev in pairs], axis=-1)

    body = body_ref[...].reshape(ROWS, 3)
    feats = jnp.concatenate([body, feats_m], axis=-1)
    hres = feats + jax.nn.relu(
        jnp.dot(feats, resw_ref[...],
                preferred_element_type=jnp.float32) + resb_ref[...])
    feats2 = jnp.concatenate([feats, hres], axis=-1)

    pooled = jnp.sum(feats2.reshape(NBATCH, N, RES_OUT), axis=1)
    pooled_ref[pid] = pooled

    @pl.when(pid == GRID - 1)
    def _():
        pall = pooled_ref[...].reshape(B, RES_OUT)
        x = jax.nn.softplus(
            jnp.dot(pall, w1_ref[...],
                    preferred_element_type=jnp.float32) + b1_ref[...])
        mean = jnp.mean(x, axis=0, keepdims=True)
        var = jnp.mean((x - mean) * (x - mean), axis=0, keepdims=True)
        xn = jax.nn.softplus(
            bng_ref[...] * (x - mean) * jax.lax.rsqrt(var + 1e-5)
            + bnb_ref[...])
        out_ref[...] = jax.nn.sigmoid(
            jnp.dot(xn, wout_ref[...],
                    preferred_element_type=jnp.float32) + bout_ref[...])


def kernel(xyz, Z, body23, emb, R_W0, R_b0, R_Wh, R_bh, R_Wf, R_bf,
           res_W, res_b, W1, b1, bn_g, bn_b, W_out, b_out):
    f32 = jnp.float32
    wf4 = R_Wf.reshape(H, CORD, CD, EMB)
    a2 = jnp.einsum('klmc,zc->kzlm', wf4, emb).reshape(H, 6 * CORD * CD)
    bemb = jnp.einsum('lmc,zc->zlm', R_bf.reshape(CORD, CD, EMB),
                      emb).reshape(6, CORD * CD)
    parts = []
    for z in range(6):
        blk = jnp.zeros((128, CORD * CD), f32)
        blk = blk.at[:H].set(0.2 * a2[:, z * 72:(z + 1) * 72])
        blk = blk.at[H].set(bemb[z])
        parts.append(blk)
    a2z = jnp.concatenate(parts, axis=0)
    w0_s = 5.0 * R_W0
    b0_s = 5.0 * R_b0
    bh_s = 5.0 * R_bh
    pad_j = [(0, 0), (0, 0), (0, NP - N)]
    zoh = jnp.pad(jax.nn.one_hot(Z, 6, dtype=f32).transpose(0, 2, 1), pad_j)
    xyz_t = jnp.pad(xyz.transpose(0, 2, 1), pad_j)

    p219 = _perm219()
    p438 = np.concatenate([p219, p219 + CLOUD_OUT])
    res_w_p = res_W[p219][:, p219]
    res_b_p = res_b[p219].reshape(1, CLOUD_OUT)
    w1_p = W1[p438, :]

    full = lambda a: pl.BlockSpec(a.shape, lambda ib: (0,) * a.ndim)
    out = pl.pallas_call(
        _se3_kernel,
        grid=(GRID,),
        in_specs=[
            pl.BlockSpec((NBATCH, 3, NP), lambda ib: (ib, 0, 0)),
            pl.BlockSpec((NBATCH, 6, NP), lambda ib: (ib, 0, 0)),
            pl.BlockSpec((NBATCH, N, 3), lambda ib: (ib, 0, 0)),
            full(w0_s),
            pl.BlockSpec((1, H), lambda ib: (0, 0)),
            full(R_Wh),
            pl.BlockSpec((LRAD - 1, 1, H), lambda ib: (0, 0, 0)),
            full(a2z),
            full(res_w_p),
            full(res_b_p),
            full(w1_p),
            pl.BlockSpec((1, FF1), lambda ib: (0, 0)),
            pl.BlockSpec((1, FF1), lambda ib: (0, 0)),
            pl.BlockSpec((1, FF1), lambda ib: (0, 0)),
            full(W_out),
            pl.BlockSpec((1, 1), lambda ib: (0, 0)),
        ],
        out_specs=pl.BlockSpec((B, 1), lambda ib: (0, 0)),
        out_shape=jax.ShapeDtypeStruct((B, 1), f32),
        scratch_shapes=[pltpu.VMEM((GRID, NBATCH, RES_OUT), f32)],
    )(xyz_t, zoh, body23,
      w0_s, b0_s.reshape(1, H), R_Wh, bh_s.reshape(LRAD - 1, 1, H),
      a2z, res_w_p, res_b_p, w1_p, b1.reshape(1, FF1),
      bn_g.reshape(1, FF1), bn_b.reshape(1, FF1), W_out,
      b_out.reshape(1, 1))
    return out
